# Initial kernel scaffold; baseline (speedup 1.0000x reference)
import functools

import jax
import jax.numpy as jnp
from jax import lax
from jax.experimental import pallas as pl
from jax.experimental.pallas import tpu as pltpu
from jax.experimental.pallas import tpu_sc as plsc

NC = 2
NS = 16
NW = NC * NS
EB = 128


def _mesh():
  return plsc.VectorSubcoreMesh(core_axis_name="c", subcore_axis_name="s")


def _make_degree_kernel(e2, npad, bpw):
  epw = bpw * EB

  @functools.partial(
      pl.kernel,
      out_type=jax.ShapeDtypeStruct((NC * npad,), jnp.float32),
      mesh=_mesh(),
      scratch_types=[
          pltpu.VMEM((EB,), jnp.int32),
          pltpu.VMEM((EB,), jnp.float32),
          pltpu.VMEM_SHARED((npad,), jnp.float32),
      ],
  )
  def deg_kernel(dst_hbm, zeros_hbm, out_hbm, didx_v, ones_v, acc_sh):
    c = lax.axis_index("c")
    s = lax.axis_index("s")
    wid = c * NS + s

    for i in range(EB // 16):
      ones_v[pl.ds(i * 16, 16)] = jnp.ones((16,), jnp.float32)

    @pl.when(s == 0)
    def _():
      pltpu.sync_copy(zeros_hbm, acc_sh)
    plsc.subcore_barrier()

    def body(b, carry):
      off = pl.multiple_of(wid * epw + b * EB, EB)
      pltpu.sync_copy(dst_hbm.at[pl.ds(off, EB)], didx_v)
      pltpu.sync_copy(ones_v, acc_sh.at[didx_v], add=True)
      return carry

    lax.fori_loop(0, bpw, body, 0)
    plsc.subcore_barrier()

    @pl.when(s == 0)
    def _():
      pltpu.sync_copy(acc_sh, out_hbm.at[pl.ds(c * npad, npad)])

  return deg_kernel


def _make_agg_kernel(e2, npad, d, bpw):
  epw = bpw * EB
  rows_per_tile = npad // NS

  @functools.partial(
      pl.kernel,
      out_type=jax.ShapeDtypeStruct((NC * npad, d), jnp.float32),
      mesh=_mesh(),
      scratch_types=[
          pltpu.VMEM((EB,), jnp.int32),
          pltpu.VMEM((EB,), jnp.int32),
          pltpu.VMEM((EB, d), jnp.float32),
          pltpu.VMEM_SHARED((npad, d), jnp.float32),
          pltpu.SemaphoreType.DMA,
      ],
  )
  def agg_kernel(xt_hbm, src_hbm, dst_hbm, zeros_hbm, out_hbm,
                 sidx_v, didx_v, rows_v, acc_sh, sem):
    c = lax.axis_index("c")
    s = lax.axis_index("s")
    wid = c * NS + s

    r0 = s * rows_per_tile
    pltpu.sync_copy(zeros_hbm.at[pl.ds(r0, rows_per_tile)],
                    acc_sh.at[pl.ds(r0, rows_per_tile)])
    plsc.subcore_barrier()

    def body(b, carry):
      off = pl.multiple_of(wid * epw + b * EB, EB)
      pltpu.sync_copy(src_hbm.at[pl.ds(off, EB)], sidx_v)
      pltpu.sync_copy(dst_hbm.at[pl.ds(off, EB)], didx_v)
      pltpu.async_copy(xt_hbm.at[sidx_v], rows_v, sem).wait()
      pltpu.sync_copy(rows_v, acc_sh.at[didx_v], add=True)
      return carry

    lax.fori_loop(0, bpw, body, 0)
    plsc.subcore_barrier()

    pltpu.sync_copy(acc_sh.at[pl.ds(r0, rows_per_tile)],
                    out_hbm.at[pl.ds(c * npad + r0, rows_per_tile)])

  return agg_kernel


def _scale_body(x_ref, d0_ref, d1_ref, o_ref):
  cnt = d0_ref[...] + d1_ref[...] + 1.0
  dis = lax.rsqrt(cnt)
  o_ref[...] = x_ref[...] * dis


def _combine_body(ya_ref, yb_ref, x_ref, d0_ref, d1_ref,
                  wc_ref, wl_ref, bc_ref, bl_ref, o_ref):
  cnt = d0_ref[...] + d1_ref[...] + 1.0
  dis = lax.rsqrt(cnt)
  y = dis * (ya_ref[...] + yb_ref[...]) + (dis * dis) * x_ref[...]
  dn = (((1,), (1,)), ((), ()))
  agg = lax.dot_general(y, wc_ref[...], dn,
                        preferred_element_type=jnp.float32) + bc_ref[...]
  o_ref[...] = lax.dot_general(agg, wl_ref[...], dn,
                               preferred_element_type=jnp.float32) + bl_ref[...]


def kernel(x, edge_index, W_conv, b_conv, W_lin, b_lin):
  n, d = x.shape
  e = edge_index.shape[1]

  npad = ((n + 31) // 32) * 32
  e2 = ((e + NW * EB - 1) // (NW * EB)) * (NW * EB)
  bpw = e2 // (NW * EB)

  src = edge_index[0]
  dst = edge_index[1]
  pad_idx = jnp.full((e2 - e,), n, dtype=jnp.int32)
  src_p = jnp.concatenate([src, pad_idx])
  dst_p = jnp.concatenate([dst, pad_idx])
  x_p = jnp.pad(x, ((0, npad - n), (0, 0)))
  zeros_1d = jnp.zeros((npad,), jnp.float32)
  zeros_2d = jnp.zeros((npad, d), jnp.float32)

  degp = _make_degree_kernel(e2, npad, bpw)(dst_p, zeros_1d)
  d0 = degp[:npad].reshape(npad, 1)
  d1 = degp[npad:].reshape(npad, 1)

  nb = 8
  br = npad // nb
  xt = pl.pallas_call(
      _scale_body,
      grid=(nb,),
      in_specs=[
          pl.BlockSpec((br, d), lambda i: (i, 0)),
          pl.BlockSpec((br, 1), lambda i: (i, 0)),
          pl.BlockSpec((br, 1), lambda i: (i, 0)),
      ],
      out_specs=pl.BlockSpec((br, d), lambda i: (i, 0)),
      out_shape=jax.ShapeDtypeStruct((npad, d), jnp.float32),
  )(x_p, d0, d1)

  y0 = _make_agg_kernel(e2, npad, d, bpw)(xt, src_p, dst_p, zeros_2d)

  h = W_conv.shape[0]
  out_p = pl.pallas_call(
      _combine_body,
      grid=(nb,),
      in_specs=[
          pl.BlockSpec((br, d), lambda i: (i, 0)),
          pl.BlockSpec((br, d), lambda i: (i, 0)),
          pl.BlockSpec((br, d), lambda i: (i, 0)),
          pl.BlockSpec((br, 1), lambda i: (i, 0)),
          pl.BlockSpec((br, 1), lambda i: (i, 0)),
          pl.BlockSpec((h, d), lambda i: (0, 0)),
          pl.BlockSpec((d, h), lambda i: (0, 0)),
          pl.BlockSpec((1, h), lambda i: (0, 0)),
          pl.BlockSpec((1, d), lambda i: (0, 0)),
      ],
      out_specs=pl.BlockSpec((br, d), lambda i: (i, 0)),
      out_shape=jax.ShapeDtypeStruct((npad, d), jnp.float32),
  )(y0[:npad], y0[npad:], x_p, d0, d1,
    W_conv, W_lin, b_conv.reshape(1, h), b_lin.reshape(1, d))

  return out_p[:n]


# speedup vs baseline: 15.3953x; 15.3953x over previous
import functools

import jax
import jax.numpy as jnp
from jax import lax
from jax.experimental import pallas as pl
from jax.experimental.pallas import tpu as pltpu
from jax.experimental.pallas import tpu_sc as plsc

NC = 2
NS = 16
NW = NC * NS
EB = 128


def _mesh():
  return plsc.VectorSubcoreMesh(core_axis_name="c", subcore_axis_name="s")


def _make_degree_kernel(e2, npad, bpw):
  epw = bpw * EB

  @functools.partial(
      pl.kernel,
      out_type=jax.ShapeDtypeStruct((NC * npad,), jnp.float32),
      mesh=_mesh(),
      scratch_types=[
          pltpu.VMEM((EB,), jnp.int32),
          pltpu.VMEM((EB,), jnp.float32),
          pltpu.VMEM_SHARED((npad,), jnp.float32),
      ],
  )
  def deg_kernel(dst_hbm, zeros_hbm, out_hbm, didx_v, ones_v, acc_sh):
    c = lax.axis_index("c")
    s = lax.axis_index("s")
    wid = c * NS + s

    for i in range(EB // 16):
      ones_v[pl.ds(i * 16, 16)] = jnp.ones((16,), jnp.float32)

    @pl.when(s == 0)
    def _():
      pltpu.sync_copy(zeros_hbm, acc_sh)
    plsc.subcore_barrier()

    def body(b, carry):
      off = pl.multiple_of(wid * epw + b * EB, EB)
      pltpu.sync_copy(dst_hbm.at[pl.ds(off, EB)], didx_v)
      pltpu.sync_copy(ones_v, acc_sh.at[didx_v], add=True)
      return carry

    lax.fori_loop(0, bpw, body, 0)
    plsc.subcore_barrier()

    @pl.when(s == 0)
    def _():
      pltpu.sync_copy(acc_sh, out_hbm.at[pl.ds(c * npad, npad)])

  return deg_kernel


def _make_agg_kernel(e2, npad, d, bpw):
  epw = bpw * EB
  rows_per_tile = npad // NS

  @functools.partial(
      pl.kernel,
      out_type=jax.ShapeDtypeStruct((NC * npad, d), jnp.float32),
      mesh=_mesh(),
      scratch_types=[
          pltpu.VMEM((EB,), jnp.int32),
          pltpu.VMEM((EB,), jnp.int32),
          pltpu.VMEM((EB, d), jnp.float32),
          pltpu.VMEM_SHARED((npad, d), jnp.float32),
          pltpu.SemaphoreType.DMA,
      ],
  )
  def agg_kernel(xt_hbm, src_hbm, dst_hbm, zeros_hbm, out_hbm,
                 sidx_v, didx_v, rows_v, acc_sh, sem):
    c = lax.axis_index("c")
    s = lax.axis_index("s")
    wid = c * NS + s

    r0 = s * rows_per_tile
    pltpu.sync_copy(zeros_hbm.at[pl.ds(r0, rows_per_tile)],
                    acc_sh.at[pl.ds(r0, rows_per_tile)])
    plsc.subcore_barrier()

    def body(b, carry):
      off = pl.multiple_of(wid * epw + b * EB, EB)
      pltpu.sync_copy(src_hbm.at[pl.ds(off, EB)], sidx_v)
      pltpu.sync_copy(dst_hbm.at[pl.ds(off, EB)], didx_v)
      pltpu.async_copy(xt_hbm.at[sidx_v], rows_v, sem).wait()
      pltpu.sync_copy(rows_v, acc_sh.at[didx_v], add=True)
      return carry

    lax.fori_loop(0, bpw, body, 0)
    plsc.subcore_barrier()

    pltpu.sync_copy(acc_sh.at[pl.ds(r0, rows_per_tile)],
                    out_hbm.at[pl.ds(c * npad + r0, rows_per_tile)])

  return agg_kernel


def _scale_body(x_ref, d0_ref, d1_ref, o_ref):
  cnt = d0_ref[...] + d1_ref[...] + 1.0
  dis = lax.rsqrt(cnt)
  o_ref[...] = x_ref[...] * dis


def _combine_body(ya_ref, yb_ref, x_ref, d0_ref, d1_ref,
                  wc_ref, wl_ref, bc_ref, bl_ref, o_ref):
  cnt = d0_ref[...] + d1_ref[...] + 1.0
  dis = lax.rsqrt(cnt)
  y = dis * (ya_ref[...] + yb_ref[...]) + (dis * dis) * x_ref[...]
  dn = (((1,), (1,)), ((), ()))
  agg = lax.dot_general(y, wc_ref[...], dn,
                        preferred_element_type=jnp.float32) + bc_ref[...]
  o_ref[...] = lax.dot_general(agg, wl_ref[...], dn,
                               preferred_element_type=jnp.float32) + bl_ref[...]


def kernel(x, edge_index, W_conv, b_conv, W_lin, b_lin):
  n, d = x.shape
  e = edge_index.shape[1]

  npad = ((n + 255) // 256) * 256
  e2 = ((e + NW * EB - 1) // (NW * EB)) * (NW * EB)
  bpw = e2 // (NW * EB)

  src = edge_index[0]
  dst = edge_index[1]
  pad_idx = jnp.full((e2 - e,), n, dtype=jnp.int32)
  src_p = jnp.concatenate([src, pad_idx])
  dst_p = jnp.concatenate([dst, pad_idx])
  x_p = jnp.pad(x, ((0, npad - n), (0, 0)))
  zeros_1d = jnp.zeros((npad,), jnp.float32)
  zeros_2d = jnp.zeros((npad, d), jnp.float32)

  degp = _make_degree_kernel(e2, npad, bpw)(dst_p, zeros_1d)
  d0 = degp[:npad].reshape(npad, 1)
  d1 = degp[npad:].reshape(npad, 1)

  nb = 8
  br = npad // nb
  xt = pl.pallas_call(
      _scale_body,
      grid=(nb,),
      in_specs=[
          pl.BlockSpec((br, d), lambda i: (i, 0)),
          pl.BlockSpec((br, 1), lambda i: (i, 0)),
          pl.BlockSpec((br, 1), lambda i: (i, 0)),
      ],
      out_specs=pl.BlockSpec((br, d), lambda i: (i, 0)),
      out_shape=jax.ShapeDtypeStruct((npad, d), jnp.float32),
  )(x_p, d0, d1)

  y0 = _make_agg_kernel(e2, npad, d, bpw)(xt, src_p, dst_p, zeros_2d)

  h = W_conv.shape[0]
  out_p = pl.pallas_call(
      _combine_body,
      grid=(nb,),
      in_specs=[
          pl.BlockSpec((br, d), lambda i: (i, 0)),
          pl.BlockSpec((br, d), lambda i: (i, 0)),
          pl.BlockSpec((br, d), lambda i: (i, 0)),
          pl.BlockSpec((br, 1), lambda i: (i, 0)),
          pl.BlockSpec((br, 1), lambda i: (i, 0)),
          pl.BlockSpec((h, d), lambda i: (0, 0)),
          pl.BlockSpec((d, h), lambda i: (0, 0)),
          pl.BlockSpec((1, h), lambda i: (0, 0)),
          pl.BlockSpec((1, d), lambda i: (0, 0)),
      ],
      out_specs=pl.BlockSpec((br, d), lambda i: (i, 0)),
      out_shape=jax.ShapeDtypeStruct((npad, d), jnp.float32),
  )(y0[:npad], y0[npad:], x_p, d0, d1,
    W_conv, W_lin, b_conv.reshape(1, h), b_lin.reshape(1, d))

  return out_p[:n]


# speedup vs baseline: 28.5210x; 1.8526x over previous
import functools

import jax
import jax.numpy as jnp
from jax import lax
from jax.experimental import pallas as pl
from jax.experimental.pallas import tpu as pltpu
from jax.experimental.pallas import tpu_sc as plsc

NC = 2
NS = 16
NW = NC * NS
EB = 128
NBUF = 3


def _mesh():
  return plsc.VectorSubcoreMesh(core_axis_name="c", subcore_axis_name="s")


def _make_degree_kernel(npad, bpw):
  epw = bpw * EB
  outer = bpw // NBUF

  @functools.partial(
      pl.kernel,
      out_type=jax.ShapeDtypeStruct((NC * npad,), jnp.float32),
      mesh=_mesh(),
      scratch_types=(
          [pltpu.VMEM((EB,), jnp.int32)] * NBUF
          + [pltpu.VMEM((EB,), jnp.float32)]
          + [pltpu.VMEM_SHARED((npad,), jnp.float32)]
          + [pltpu.SemaphoreType.DMA] * NBUF
      ),
  )
  def deg_kernel(dst_hbm, zeros_hbm, out_hbm, *scr):
    didx = list(scr[:NBUF])
    ones_v = scr[NBUF]
    acc_sh = scr[NBUF + 1]
    sems = list(scr[NBUF + 2:])

    c = lax.axis_index("c")
    s = lax.axis_index("s")
    wid = c * NS + s
    base = wid * epw

    for i in range(EB // 16):
      ones_v[pl.ds(i * 16, 16)] = jnp.ones((16,), jnp.float32)

    @pl.when(s == 0)
    def _():
      pltpu.sync_copy(zeros_hbm, acc_sh)
    plsc.subcore_barrier()

    for j in range(NBUF):
      off = pl.multiple_of(base + j * EB, EB)
      pltpu.sync_copy(dst_hbm.at[pl.ds(off, EB)], didx[j])
      pltpu.async_copy(ones_v, acc_sh.at[didx[j]], sems[j], add=True)

    def body(t, carry):
      for j in range(NBUF):
        b = (t + 1) * NBUF + j
        pltpu.make_async_copy(ones_v, acc_sh.at[didx[j]], sems[j]).wait()
        off = pl.multiple_of(base + b * EB, EB)
        pltpu.sync_copy(dst_hbm.at[pl.ds(off, EB)], didx[j])
        pltpu.async_copy(ones_v, acc_sh.at[didx[j]], sems[j], add=True)
      return carry

    lax.fori_loop(0, outer - 1, body, 0)
    for j in range(NBUF):
      pltpu.make_async_copy(ones_v, acc_sh.at[didx[j]], sems[j]).wait()
    plsc.subcore_barrier()

    @pl.when(s == 0)
    def _():
      pltpu.sync_copy(acc_sh, out_hbm.at[pl.ds(c * npad, npad)])

  return deg_kernel


def _make_agg_kernel(npad, d, bpw):
  epw = bpw * EB
  outer = bpw // NBUF
  rows_per_tile = npad // NS

  @functools.partial(
      pl.kernel,
      out_type=jax.ShapeDtypeStruct((NC * npad, d), jnp.float32),
      mesh=_mesh(),
      scratch_types=(
          [pltpu.VMEM((EB,), jnp.int32)] * NBUF
          + [pltpu.VMEM((EB,), jnp.int32)] * NBUF
          + [pltpu.VMEM((EB, d), jnp.float32)] * NBUF
          + [pltpu.VMEM_SHARED((npad, d), jnp.float32)]
          + [pltpu.SemaphoreType.DMA] * (2 * NBUF)
      ),
  )
  def agg_kernel(xt_hbm, src_hbm, dst_hbm, zeros_hbm, out_hbm, *scr):
    sidx = list(scr[:NBUF])
    didx = list(scr[NBUF:2 * NBUF])
    rows = list(scr[2 * NBUF:3 * NBUF])
    acc_sh = scr[3 * NBUF]
    semg = list(scr[3 * NBUF + 1:3 * NBUF + 1 + NBUF])
    sems = list(scr[3 * NBUF + 1 + NBUF:])

    c = lax.axis_index("c")
    s = lax.axis_index("s")
    wid = c * NS + s
    base = wid * epw

    r0 = s * rows_per_tile
    pltpu.sync_copy(zeros_hbm.at[pl.ds(r0, rows_per_tile)],
                    acc_sh.at[pl.ds(r0, rows_per_tile)])
    plsc.subcore_barrier()

    for j in range(NBUF):
      off = pl.multiple_of(base + j * EB, EB)
      pltpu.sync_copy(src_hbm.at[pl.ds(off, EB)], sidx[j])
      pltpu.sync_copy(dst_hbm.at[pl.ds(off, EB)], didx[j])
      pltpu.async_copy(xt_hbm.at[sidx[j]], rows[j], semg[j])

    def body(t, carry):
      for j in range(NBUF):
        b = t * NBUF + j
        pltpu.make_async_copy(xt_hbm.at[sidx[j]], rows[j], semg[j]).wait()
        pltpu.async_copy(rows[j], acc_sh.at[didx[j]], sems[j], add=True)
        pltpu.make_async_copy(rows[j], acc_sh.at[didx[j]], sems[j]).wait()
        off = pl.multiple_of(base + (b + NBUF) * EB, EB)
        pltpu.sync_copy(src_hbm.at[pl.ds(off, EB)], sidx[j])
        pltpu.sync_copy(dst_hbm.at[pl.ds(off, EB)], didx[j])
        pltpu.async_copy(xt_hbm.at[sidx[j]], rows[j], semg[j])
      return carry

    lax.fori_loop(0, outer - 1, body, 0)

    for j in range(NBUF):
      pltpu.make_async_copy(xt_hbm.at[sidx[j]], rows[j], semg[j]).wait()
      pltpu.async_copy(rows[j], acc_sh.at[didx[j]], sems[j], add=True)
    for j in range(NBUF):
      pltpu.make_async_copy(rows[j], acc_sh.at[didx[j]], sems[j]).wait()
    plsc.subcore_barrier()

    pltpu.sync_copy(acc_sh.at[pl.ds(r0, rows_per_tile)],
                    out_hbm.at[pl.ds(c * npad + r0, rows_per_tile)])

  return agg_kernel


def _scale_body(x_ref, d0_ref, d1_ref, o_ref):
  cnt = d0_ref[...] + d1_ref[...] + 1.0
  dis = lax.rsqrt(cnt)
  o_ref[...] = x_ref[...] * dis


def _combine_body(ya_ref, yb_ref, x_ref, d0_ref, d1_ref,
                  wc_ref, wl_ref, bc_ref, bl_ref, o_ref):
  cnt = d0_ref[...] + d1_ref[...] + 1.0
  dis = lax.rsqrt(cnt)
  y = dis * (ya_ref[...] + yb_ref[...]) + (dis * dis) * x_ref[...]
  dn = (((1,), (1,)), ((), ()))
  agg = lax.dot_general(y, wc_ref[...], dn,
                        preferred_element_type=jnp.float32) + bc_ref[...]
  o_ref[...] = lax.dot_general(agg, wl_ref[...], dn,
                               preferred_element_type=jnp.float32) + bl_ref[...]


def kernel(x, edge_index, W_conv, b_conv, W_lin, b_lin):
  n, d = x.shape
  e = edge_index.shape[1]

  npad = ((n + 127) // 128) * 128
  npad1 = npad
  chunk = NW * EB * NBUF
  e2 = ((e + chunk - 1) // chunk) * chunk
  bpw = e2 // (NW * EB)

  src = edge_index[0]
  dst = edge_index[1]
  pad_r = jnp.arange(e2 - e, dtype=jnp.int32)
  src_p = jnp.concatenate([src, n + pad_r % (npad - n)])
  dst_p = jnp.concatenate([dst, n + pad_r % (npad - n)])
  x_p = jnp.pad(x, ((0, npad - n), (0, 0)))
  zeros_1d = jnp.zeros((npad1,), jnp.float32)
  zeros_2d = jnp.zeros((npad, d), jnp.float32)

  degp = _make_degree_kernel(npad1, bpw)(dst_p, zeros_1d)
  d0 = degp[:npad].reshape(npad, 1)
  d1 = degp[npad1:npad1 + npad].reshape(npad, 1)

  nb = 4
  br = npad // nb
  xt = pl.pallas_call(
      _scale_body,
      grid=(nb,),
      in_specs=[
          pl.BlockSpec((br, d), lambda i: (i, 0)),
          pl.BlockSpec((br, 1), lambda i: (i, 0)),
          pl.BlockSpec((br, 1), lambda i: (i, 0)),
      ],
      out_specs=pl.BlockSpec((br, d), lambda i: (i, 0)),
      out_shape=jax.ShapeDtypeStruct((npad, d), jnp.float32),
  )(x_p, d0, d1)

  y0 = _make_agg_kernel(npad, d, bpw)(xt, src_p, dst_p, zeros_2d)

  h = W_conv.shape[0]
  out_p = pl.pallas_call(
      _combine_body,
      grid=(nb,),
      in_specs=[
          pl.BlockSpec((br, d), lambda i: (i, 0)),
          pl.BlockSpec((br, d), lambda i: (i, 0)),
          pl.BlockSpec((br, d), lambda i: (i, 0)),
          pl.BlockSpec((br, 1), lambda i: (i, 0)),
          pl.BlockSpec((br, 1), lambda i: (i, 0)),
          pl.BlockSpec((h, d), lambda i: (0, 0)),
          pl.BlockSpec((d, h), lambda i: (0, 0)),
          pl.BlockSpec((1, h), lambda i: (0, 0)),
          pl.BlockSpec((1, d), lambda i: (0, 0)),
      ],
      out_specs=pl.BlockSpec((br, d), lambda i: (i, 0)),
      out_shape=jax.ShapeDtypeStruct((npad, d), jnp.float32),
  )(y0[:npad], y0[npad:], x_p, d0, d1,
    W_conv, W_lin, b_conv.reshape(1, h), b_lin.reshape(1, d))

  return out_p[:n]
